# Initial kernel scaffold; baseline (speedup 1.0000x reference)
import jax
import jax.numpy as jnp
from jax.experimental import pallas as pl


def kernel(x, edge_index, batch, Ws1, bs1, Ws2, bs2, jk_W, jk_b, c1_W, c1_b, c2_W, c2_b):
    raise NotImplementedError("write your pallas kernel here")


# speedup vs baseline: 5.6950x; 5.6950x over previous
import functools

import jax
import jax.numpy as jnp
from jax import lax
from jax.experimental import pallas as pl
from jax.experimental.pallas import tpu as pltpu
from jax.experimental.pallas import tpu_sc as plsc

_N = 10000
_E = 320000
_D = 128
_L = 5
_G = 128
_OUT = 16

_NC = 2
_NS = 16
_NW = _NC * _NS
_EPW = _E // _NW
_CH = 128
_NFULL = _EPW // _CH
_TAIL = _EPW - _NFULL * _CH
_NPAD = 10240
_RPT = _NPAD // _NS
_ZR = 128

_mesh = plsc.VectorSubcoreMesh(core_axis_name="c", subcore_axis_name="s")


@functools.partial(
    pl.kernel,
    mesh=_mesh,
    out_type=jax.ShapeDtypeStruct((_NC, _NPAD, _D), jnp.float32),
    scratch_types=[
        pltpu.VMEM((_CH,), jnp.int32),
        pltpu.VMEM((_CH,), jnp.int32),
        pltpu.VMEM((_CH, _D), jnp.float32),
        pltpu.VMEM((_TAIL,), jnp.int32),
        pltpu.VMEM((_TAIL,), jnp.int32),
        pltpu.VMEM((_TAIL, _D), jnp.float32),
        pltpu.VMEM((_ZR, _D), jnp.float32),
        pltpu.VMEM_SHARED((_NPAD, _D), jnp.float32),
        pltpu.SemaphoreType.DMA,
    ],
)
def _seg_sum_sc(h_hbm, src_hbm, dst_hbm, out_hbm,
                sidx, didx, rows, sidx_t, didx_t, rows_t, zbuf, acc, sem):
    cid = lax.axis_index("c")
    sid = lax.axis_index("s")
    wid = sid * _NC + cid

    zero = jnp.zeros((16,), jnp.float32)

    def _zrow(r, carry):
        for c in range(_D // 16):
            zbuf[r, pl.ds(c * 16, 16)] = zero
        return carry

    lax.fori_loop(0, _ZR, _zrow, 0)
    for k in range(_RPT // _ZR):
        pltpu.sync_copy(zbuf, acc.at[pl.ds(sid * _RPT + k * _ZR, _ZR)])
    plsc.subcore_barrier()

    base = wid * _EPW

    def _chunk(i, carry):
        off = base + i * _CH
        pltpu.sync_copy(src_hbm.at[pl.ds(off, _CH)], sidx)
        pltpu.sync_copy(dst_hbm.at[pl.ds(off, _CH)], didx)
        pltpu.async_copy(h_hbm.at[sidx], rows, sem).wait()
        pltpu.sync_copy(rows, acc.at[didx], add=True)
        return carry

    lax.fori_loop(0, _NFULL, _chunk, 0)

    offt = base + _NFULL * _CH
    pltpu.sync_copy(src_hbm.at[pl.ds(offt, _TAIL)], sidx_t)
    pltpu.sync_copy(dst_hbm.at[pl.ds(offt, _TAIL)], didx_t)
    pltpu.async_copy(h_hbm.at[sidx_t], rows_t, sem).wait()
    pltpu.sync_copy(rows_t, acc.at[didx_t], add=True)

    plsc.subcore_barrier()
    pltpu.sync_copy(acc.at[pl.ds(sid * _RPT, _RPT)],
                    out_hbm.at[cid, pl.ds(sid * _RPT, _RPT)])


_MB = 2000
_NMB = _N // _MB


def _mlp_body(h_ref, a0_ref, a1_ref, w1_ref, b1_ref, w2_ref, b2_ref, o_ref):
    z = h_ref[...] + a0_ref[0] + a1_ref[0]
    z = jnp.dot(z, w1_ref[...], preferred_element_type=jnp.float32) + b1_ref[...]
    z = jnp.maximum(z, 0.0)
    z = jnp.dot(z, w2_ref[...], preferred_element_type=jnp.float32) + b2_ref[...]
    o_ref[...] = jnp.maximum(z, 0.0)


def _mlp(h, agg2, w1, b1, w2, b2):
    return pl.pallas_call(
        _mlp_body,
        grid=(_NMB,),
        in_specs=[
            pl.BlockSpec((_MB, _D), lambda i: (i, 0)),
            pl.BlockSpec((1, _MB, _D), lambda i: (0, i, 0)),
            pl.BlockSpec((1, _MB, _D), lambda i: (1, i, 0)),
            pl.BlockSpec((_D, _D), lambda i: (0, 0)),
            pl.BlockSpec((1, _D), lambda i: (0, 0)),
            pl.BlockSpec((_D, _D), lambda i: (0, 0)),
            pl.BlockSpec((1, _D), lambda i: (0, 0)),
        ],
        out_specs=pl.BlockSpec((_MB, _D), lambda i: (i, 0)),
        out_shape=jax.ShapeDtypeStruct((_N, _D), jnp.float32),
    )(h, agg2, agg2, w1, b1.reshape(1, _D), w2, b2.reshape(1, _D))


_PB = 2000
_NPB = _N // _PB
_SCALE = 1.0 / (1.0 + 1e-5) ** 0.5


def _final_body(h0_ref, h1_ref, h2_ref, h3_ref, h4_ref, b_ref, jkW_ref,
                jkb_ref, c1W_ref, c1b_ref, c2W_ref, c2b_ref, o_ref, pooled):
    i = pl.program_id(0)

    @pl.when(i == 0)
    def _init():
        pooled[...] = jnp.zeros_like(pooled)

    hs = (h0_ref, h1_ref, h2_ref, h3_ref, h4_ref)
    hlin = jnp.dot(hs[0][...], jkW_ref[0], preferred_element_type=jnp.float32)
    for l in range(1, _L):
        hlin += jnp.dot(hs[l][...], jkW_ref[l], preferred_element_type=jnp.float32)
    hlin += jkb_ref[...]

    bvec = b_ref[0, 0, :]
    mask = (bvec[None, :] == lax.broadcasted_iota(jnp.int32, (_G, _PB), 0))
    pooled[...] += jnp.dot(mask.astype(jnp.float32), hlin,
                           preferred_element_type=jnp.float32)

    p = pooled[...]
    c = jnp.maximum((jnp.dot(p, c1W_ref[...], preferred_element_type=jnp.float32)
                     + c1b_ref[...]) * _SCALE, 0.0)
    o_ref[...] = jnp.dot(c, c2W_ref[...], preferred_element_type=jnp.float32) + c2b_ref[...]


def _final(hs, batch3, jkWs, jk_b, c1_W, c1_b, c2_W, c2_b):
    hspec = pl.BlockSpec((_PB, _D), lambda i: (i, 0))
    return pl.pallas_call(
        _final_body,
        grid=(_NPB,),
        in_specs=[
            hspec, hspec, hspec, hspec, hspec,
            pl.BlockSpec((1, 1, _PB), lambda i: (i, 0, 0)),
            pl.BlockSpec((_L, _D, _D), lambda i: (0, 0, 0)),
            pl.BlockSpec((1, _D), lambda i: (0, 0)),
            pl.BlockSpec((_D, _D), lambda i: (0, 0)),
            pl.BlockSpec((1, _D), lambda i: (0, 0)),
            pl.BlockSpec((_D, _OUT), lambda i: (0, 0)),
            pl.BlockSpec((1, _OUT), lambda i: (0, 0)),
        ],
        out_specs=pl.BlockSpec((_G, _OUT), lambda i: (0, 0)),
        out_shape=jax.ShapeDtypeStruct((_G, _OUT), jnp.float32),
        scratch_shapes=[pltpu.VMEM((_G, _D), jnp.float32)],
    )(*hs, batch3, jkWs, jk_b.reshape(1, _D), c1_W, c1_b.reshape(1, _D),
      c2_W, c2_b.reshape(1, _OUT))


def kernel(x, edge_index, batch, Ws1, bs1, Ws2, bs2, jk_W, jk_b,
           c1_W, c1_b, c2_W, c2_b):
    src = edge_index[0]
    dst = edge_index[1]
    h = x
    hs = []
    for i in range(_L):
        agg2 = _seg_sum_sc(h, src, dst)
        h = _mlp(h, agg2, Ws1[i], bs1[i], Ws2[i], bs2[i])
        hs.append(h)
    jkWs = jk_W.reshape(_L, _D, _D)
    batch3 = batch.reshape(_NPB, 1, _PB)
    return _final(hs, batch3, jkWs, jk_b, c1_W, c1_b, c2_W, c2_b)


# speedup vs baseline: 12.0912x; 2.1231x over previous
import functools

import jax
import jax.numpy as jnp
from jax import lax
from jax.experimental import pallas as pl
from jax.experimental.pallas import tpu as pltpu
from jax.experimental.pallas import tpu_sc as plsc

_N = 10000
_E = 320000
_D = 128
_L = 5
_G = 128
_OUT = 16

_NC = 2
_NS = 16
_NW = _NC * _NS
_EPW = _E // _NW
_CH = 128
_CPS = 1
_SUP = _CPS * _CH
_NSUP = _EPW // _SUP
_TAIL = _EPW - _NSUP * _SUP
_NPAD = 10240
_RPT = _NPAD // _NS
_ZR = 32

_mesh = plsc.VectorSubcoreMesh(core_axis_name="c", subcore_axis_name="s")


@functools.partial(
    pl.kernel,
    mesh=_mesh,
    out_type=jax.ShapeDtypeStruct((_NC, _NPAD, _D), jnp.float32),
    scratch_types=[
        pltpu.VMEM((3, _CPS, _CH), jnp.int32),
        pltpu.VMEM((3, _CPS, _CH), jnp.int32),
        pltpu.VMEM((2, _SUP, _D), jnp.float32),
        pltpu.VMEM((_TAIL,), jnp.int32),
        pltpu.VMEM((_TAIL,), jnp.int32),
        pltpu.VMEM((_TAIL, _D), jnp.float32),
        pltpu.VMEM((_ZR, _D), jnp.float32),
        pltpu.VMEM_SHARED((_NPAD, _D), jnp.float32),
        pltpu.SemaphoreType.DMA,
        pltpu.SemaphoreType.DMA,
        pltpu.SemaphoreType.DMA,
    ],
)
def _seg_sum_sc(h_hbm, src_hbm, dst_hbm, out_hbm,
                sidx, didx, rows, sidx_t, didx_t, rows_t, zbuf, acc,
                sem_i, sem_g, sem_s):
    cid = lax.axis_index("c")
    sid = lax.axis_index("s")
    wid = sid * _NC + cid
    base = wid * _EPW

    zero = jnp.zeros((16,), jnp.float32)

    def _zrow(r, carry):
        for c in range(_D // 16):
            zbuf[r, pl.ds(c * 16, 16)] = zero
        return carry

    lax.fori_loop(0, _ZR, _zrow, 0)
    for k in range(_RPT // _ZR):
        pltpu.sync_copy(zbuf, acc.at[pl.ds(sid * _RPT + k * _ZR, _ZR)])
    plsc.subcore_barrier()

    def _fire_idx(s):
        b = s % 3
        off = base + s * _SUP
        for j in range(_CPS):
            pltpu.async_copy(src_hbm.at[pl.ds(off + j * _CH, _CH)],
                             sidx.at[b, j], sem_i)
            pltpu.async_copy(dst_hbm.at[pl.ds(off + j * _CH, _CH)],
                             didx.at[b, j], sem_i)

    def _wait_idx(s):
        b = s % 3
        off = base + s * _SUP
        for j in range(_CPS):
            pltpu.make_async_copy(src_hbm.at[pl.ds(off + j * _CH, _CH)],
                                  sidx.at[b, j], sem_i).wait()
            pltpu.make_async_copy(dst_hbm.at[pl.ds(off + j * _CH, _CH)],
                                  didx.at[b, j], sem_i).wait()

    def _fire_gather(s):
        b3, b2 = s % 3, s % 2
        for j in range(_CPS):
            pltpu.async_copy(h_hbm.at[sidx.at[b3, j]],
                             rows.at[b2, pl.ds(j * _CH, _CH)], sem_g)

    def _wait_gather(s):
        b3, b2 = s % 3, s % 2
        for j in range(_CPS):
            pltpu.make_async_copy(h_hbm.at[sidx.at[b3, j]],
                                  rows.at[b2, pl.ds(j * _CH, _CH)],
                                  sem_g).wait()

    def _fire_scatter(s):
        b3, b2 = s % 3, s % 2
        for j in range(_CPS):
            pltpu.async_copy(rows.at[b2, pl.ds(j * _CH, _CH)],
                             acc.at[didx.at[b3, j]], sem_s, add=True)

    def _wait_scatter(s):
        b3, b2 = s % 3, s % 2
        for j in range(_CPS):
            pltpu.make_async_copy(rows.at[b2, pl.ds(j * _CH, _CH)],
                                  acc.at[didx.at[b3, j]], sem_s).wait()

    _fire_idx(0)
    _fire_idx(1)
    _wait_idx(0)
    _fire_gather(0)

    def _body(s, carry):
        @pl.when(s >= 1)
        def _():
            _wait_scatter(s - 1)

        @pl.when(s + 2 <= _NSUP - 1)
        def _():
            _fire_idx(s + 2)

        @pl.when(s + 1 <= _NSUP - 1)
        def _():
            _wait_idx(s + 1)
            _fire_gather(s + 1)

        _wait_gather(s)
        _fire_scatter(s)
        return carry

    lax.fori_loop(0, _NSUP, _body, 0)
    _wait_scatter(_NSUP - 1)

    offt = base + _NSUP * _SUP
    pltpu.sync_copy(src_hbm.at[pl.ds(offt, _TAIL)], sidx_t)
    pltpu.sync_copy(dst_hbm.at[pl.ds(offt, _TAIL)], didx_t)
    pltpu.async_copy(h_hbm.at[sidx_t], rows_t, sem_g).wait()
    pltpu.sync_copy(rows_t, acc.at[didx_t], add=True)

    plsc.subcore_barrier()
    pltpu.sync_copy(acc.at[pl.ds(sid * _RPT, _RPT)],
                    out_hbm.at[cid, pl.ds(sid * _RPT, _RPT)])


_MB = 2000
_NMB = _N // _MB


def _mlp_body(h_ref, a0_ref, a1_ref, w1_ref, b1_ref, w2_ref, b2_ref, o_ref):
    z = h_ref[...] + a0_ref[0] + a1_ref[0]
    z = jnp.dot(z, w1_ref[...], preferred_element_type=jnp.float32) + b1_ref[...]
    z = jnp.maximum(z, 0.0)
    z = jnp.dot(z, w2_ref[...], preferred_element_type=jnp.float32) + b2_ref[...]
    o_ref[...] = jnp.maximum(z, 0.0)


def _mlp(h, agg2, w1, b1, w2, b2):
    return pl.pallas_call(
        _mlp_body,
        grid=(_NMB,),
        in_specs=[
            pl.BlockSpec((_MB, _D), lambda i: (i, 0)),
            pl.BlockSpec((1, _MB, _D), lambda i: (0, i, 0)),
            pl.BlockSpec((1, _MB, _D), lambda i: (1, i, 0)),
            pl.BlockSpec((_D, _D), lambda i: (0, 0)),
            pl.BlockSpec((1, _D), lambda i: (0, 0)),
            pl.BlockSpec((_D, _D), lambda i: (0, 0)),
            pl.BlockSpec((1, _D), lambda i: (0, 0)),
        ],
        out_specs=pl.BlockSpec((_MB, _D), lambda i: (i, 0)),
        out_shape=jax.ShapeDtypeStruct((_N, _D), jnp.float32),
    )(h, agg2, agg2, w1, b1.reshape(1, _D), w2, b2.reshape(1, _D))


_PB = 2000
_NPB = _N // _PB
_SCALE = 1.0 / (1.0 + 1e-5) ** 0.5


def _final_body(h0_ref, h1_ref, h2_ref, h3_ref, h4_ref, b_ref, jkW_ref,
                jkb_ref, c1W_ref, c1b_ref, c2W_ref, c2b_ref, o_ref, pooled):
    i = pl.program_id(0)

    @pl.when(i == 0)
    def _init():
        pooled[...] = jnp.zeros_like(pooled)

    hs = (h0_ref, h1_ref, h2_ref, h3_ref, h4_ref)
    hlin = jnp.dot(hs[0][...], jkW_ref[0], preferred_element_type=jnp.float32)
    for l in range(1, _L):
        hlin += jnp.dot(hs[l][...], jkW_ref[l], preferred_element_type=jnp.float32)
    hlin += jkb_ref[...]

    bvec = b_ref[0, 0, :]
    mask = (bvec[None, :] == lax.broadcasted_iota(jnp.int32, (_G, _PB), 0))
    pooled[...] += jnp.dot(mask.astype(jnp.float32), hlin,
                           preferred_element_type=jnp.float32)

    p = pooled[...]
    c = jnp.maximum((jnp.dot(p, c1W_ref[...], preferred_element_type=jnp.float32)
                     + c1b_ref[...]) * _SCALE, 0.0)
    o_ref[...] = jnp.dot(c, c2W_ref[...], preferred_element_type=jnp.float32) + c2b_ref[...]


def _final(hs, batch3, jkWs, jk_b, c1_W, c1_b, c2_W, c2_b):
    hspec = pl.BlockSpec((_PB, _D), lambda i: (i, 0))
    return pl.pallas_call(
        _final_body,
        grid=(_NPB,),
        in_specs=[
            hspec, hspec, hspec, hspec, hspec,
            pl.BlockSpec((1, 1, _PB), lambda i: (i, 0, 0)),
            pl.BlockSpec((_L, _D, _D), lambda i: (0, 0, 0)),
            pl.BlockSpec((1, _D), lambda i: (0, 0)),
            pl.BlockSpec((_D, _D), lambda i: (0, 0)),
            pl.BlockSpec((1, _D), lambda i: (0, 0)),
            pl.BlockSpec((_D, _OUT), lambda i: (0, 0)),
            pl.BlockSpec((1, _OUT), lambda i: (0, 0)),
        ],
        out_specs=pl.BlockSpec((_G, _OUT), lambda i: (0, 0)),
        out_shape=jax.ShapeDtypeStruct((_G, _OUT), jnp.float32),
        scratch_shapes=[pltpu.VMEM((_G, _D), jnp.float32)],
    )(*hs, batch3, jkWs, jk_b.reshape(1, _D), c1_W, c1_b.reshape(1, _D),
      c2_W, c2_b.reshape(1, _OUT))


def kernel(x, edge_index, batch, Ws1, bs1, Ws2, bs2, jk_W, jk_b,
           c1_W, c1_b, c2_W, c2_b):
    src = edge_index[0]
    dst = edge_index[1]
    h = x
    hs = []
    for i in range(_L):
        agg2 = _seg_sum_sc(h, src, dst)
        h = _mlp(h, agg2, Ws1[i], bs1[i], Ws2[i], bs2[i])
        hs.append(h)
    jkWs = jk_W.reshape(_L, _D, _D)
    batch3 = batch.reshape(_NPB, 1, _PB)
    return _final(hs, batch3, jkWs, jk_b, c1_W, c1_b, c2_W, c2_b)


# speedup vs baseline: 12.3333x; 1.0200x over previous
import functools

import jax
import jax.numpy as jnp
from jax import lax
from jax.experimental import pallas as pl
from jax.experimental.pallas import tpu as pltpu
from jax.experimental.pallas import tpu_sc as plsc

_N = 10000
_E = 320000
_D = 128
_L = 5
_G = 128
_OUT = 16

_NC = 2
_NS = 16
_NW = _NC * _NS
_EPW = _E // _NW
_CH = 128
_CPS = 1
_SUP = _CPS * _CH
_NSUP = _EPW // _SUP
_TAIL = _EPW - _NSUP * _SUP
_NPAD = 10240
_RPT = _NPAD // _NS
_ZR = 80

_mesh = plsc.VectorSubcoreMesh(core_axis_name="c", subcore_axis_name="s")


@functools.partial(
    pl.kernel,
    mesh=_mesh,
    out_type=jax.ShapeDtypeStruct((_NC, _NPAD, _D), jnp.float32),
    scratch_types=[
        pltpu.VMEM((3, _CPS, _CH), jnp.int32),
        pltpu.VMEM((3, _CPS, _CH), jnp.int32),
        pltpu.VMEM((2, _SUP, _D), jnp.float32),
        pltpu.VMEM((_TAIL,), jnp.int32),
        pltpu.VMEM((_TAIL,), jnp.int32),
        pltpu.VMEM((_TAIL, _D), jnp.float32),
        pltpu.VMEM((_ZR, _D), jnp.float32),
        pltpu.VMEM_SHARED((_NPAD, _D), jnp.float32),
        pltpu.SemaphoreType.DMA,
        pltpu.SemaphoreType.DMA,
        pltpu.SemaphoreType.DMA,
    ],
)
def _seg_sum_sc(h_hbm, src_hbm, dst_hbm, out_hbm,
                sidx, didx, rows, sidx_t, didx_t, rows_t, zbuf, acc,
                sem_i, sem_g, sem_s):
    cid = lax.axis_index("c")
    sid = lax.axis_index("s")
    wid = sid * _NC + cid
    base = wid * _EPW

    def _fire_idx(s):
        b = s % 3
        off = base + s * _SUP
        for j in range(_CPS):
            pltpu.async_copy(src_hbm.at[pl.ds(off + j * _CH, _CH)],
                             sidx.at[b, j], sem_i)
            pltpu.async_copy(dst_hbm.at[pl.ds(off + j * _CH, _CH)],
                             didx.at[b, j], sem_i)

    def _wait_idx(s):
        b = s % 3
        off = base + s * _SUP
        for j in range(_CPS):
            pltpu.make_async_copy(src_hbm.at[pl.ds(off + j * _CH, _CH)],
                                  sidx.at[b, j], sem_i).wait()
            pltpu.make_async_copy(dst_hbm.at[pl.ds(off + j * _CH, _CH)],
                                  didx.at[b, j], sem_i).wait()

    def _fire_gather(s):
        b3, b2 = s % 3, s % 2
        for j in range(_CPS):
            pltpu.async_copy(h_hbm.at[sidx.at[b3, j]],
                             rows.at[b2, pl.ds(j * _CH, _CH)], sem_g)

    def _wait_gather(s):
        b3, b2 = s % 3, s % 2
        for j in range(_CPS):
            pltpu.make_async_copy(h_hbm.at[sidx.at[b3, j]],
                                  rows.at[b2, pl.ds(j * _CH, _CH)],
                                  sem_g).wait()

    def _fire_scatter(s):
        b3, b2 = s % 3, s % 2
        for j in range(_CPS):
            pltpu.async_copy(rows.at[b2, pl.ds(j * _CH, _CH)],
                             acc.at[didx.at[b3, j]], sem_s, add=True)

    def _wait_scatter(s):
        b3, b2 = s % 3, s % 2
        for j in range(_CPS):
            pltpu.make_async_copy(rows.at[b2, pl.ds(j * _CH, _CH)],
                                  acc.at[didx.at[b3, j]], sem_s).wait()

    _fire_idx(0)
    _fire_idx(1)

    zero = jnp.zeros((16,), jnp.float32)

    def _zrow(r, carry):
        for c in range(_D // 16):
            zbuf[r, pl.ds(c * 16, 16)] = zero
        return carry

    lax.fori_loop(0, _ZR, _zrow, 0)
    for k in range(_RPT // _ZR):
        pltpu.async_copy(zbuf, acc.at[pl.ds(sid * _RPT + k * _ZR, _ZR)], sem_s)

    _wait_idx(0)
    _fire_gather(0)
    for k in range(_RPT // _ZR):
        pltpu.make_async_copy(zbuf, acc.at[pl.ds(sid * _RPT + k * _ZR, _ZR)],
                              sem_s).wait()
    plsc.subcore_barrier()

    def _body(s, carry):
        @pl.when(s >= 1)
        def _():
            _wait_scatter(s - 1)

        @pl.when(s + 2 <= _NSUP - 1)
        def _():
            _fire_idx(s + 2)

        @pl.when(s + 1 <= _NSUP - 1)
        def _():
            _wait_idx(s + 1)
            _fire_gather(s + 1)

        _wait_gather(s)
        _fire_scatter(s)
        return carry

    lax.fori_loop(0, _NSUP, _body, 0)
    _wait_scatter(_NSUP - 1)

    offt = base + _NSUP * _SUP
    pltpu.sync_copy(src_hbm.at[pl.ds(offt, _TAIL)], sidx_t)
    pltpu.sync_copy(dst_hbm.at[pl.ds(offt, _TAIL)], didx_t)
    pltpu.async_copy(h_hbm.at[sidx_t], rows_t, sem_g).wait()
    pltpu.sync_copy(rows_t, acc.at[didx_t], add=True)

    plsc.subcore_barrier()
    pltpu.sync_copy(acc.at[pl.ds(sid * _RPT, _RPT)],
                    out_hbm.at[cid, pl.ds(sid * _RPT, _RPT)])


_MB = 2000
_NMB = _N // _MB


def _mlp_body(h_ref, a0_ref, a1_ref, w1_ref, b1_ref, w2_ref, b2_ref, o_ref):
    z = h_ref[...] + a0_ref[0] + a1_ref[0]
    z = jnp.dot(z, w1_ref[...], preferred_element_type=jnp.float32) + b1_ref[...]
    z = jnp.maximum(z, 0.0)
    z = jnp.dot(z, w2_ref[...], preferred_element_type=jnp.float32) + b2_ref[...]
    o_ref[...] = jnp.maximum(z, 0.0)


def _mlp(h, agg2, w1, b1, w2, b2):
    return pl.pallas_call(
        _mlp_body,
        grid=(_NMB,),
        in_specs=[
            pl.BlockSpec((_MB, _D), lambda i: (i, 0)),
            pl.BlockSpec((1, _MB, _D), lambda i: (0, i, 0)),
            pl.BlockSpec((1, _MB, _D), lambda i: (1, i, 0)),
            pl.BlockSpec((_D, _D), lambda i: (0, 0)),
            pl.BlockSpec((1, _D), lambda i: (0, 0)),
            pl.BlockSpec((_D, _D), lambda i: (0, 0)),
            pl.BlockSpec((1, _D), lambda i: (0, 0)),
        ],
        out_specs=pl.BlockSpec((_MB, _D), lambda i: (i, 0)),
        out_shape=jax.ShapeDtypeStruct((_N, _D), jnp.float32),
    )(h, agg2, agg2, w1, b1.reshape(1, _D), w2, b2.reshape(1, _D))


_PB = 2000
_NPB = _N // _PB
_SCALE = 1.0 / (1.0 + 1e-5) ** 0.5


def _final_body(h0_ref, h1_ref, h2_ref, h3_ref, h4_ref, b_ref, jkW_ref,
                jkb_ref, c1W_ref, c1b_ref, c2W_ref, c2b_ref, o_ref, pooled):
    i = pl.program_id(0)

    @pl.when(i == 0)
    def _init():
        pooled[...] = jnp.zeros_like(pooled)

    hs = (h0_ref, h1_ref, h2_ref, h3_ref, h4_ref)
    hlin = jnp.dot(hs[0][...], jkW_ref[0], preferred_element_type=jnp.float32)
    for l in range(1, _L):
        hlin += jnp.dot(hs[l][...], jkW_ref[l], preferred_element_type=jnp.float32)
    hlin += jkb_ref[...]

    bvec = b_ref[0, 0, :]
    mask = (bvec[None, :] == lax.broadcasted_iota(jnp.int32, (_G, _PB), 0))
    pooled[...] += jnp.dot(mask.astype(jnp.float32), hlin,
                           preferred_element_type=jnp.float32)

    p = pooled[...]
    c = jnp.maximum((jnp.dot(p, c1W_ref[...], preferred_element_type=jnp.float32)
                     + c1b_ref[...]) * _SCALE, 0.0)
    o_ref[...] = jnp.dot(c, c2W_ref[...], preferred_element_type=jnp.float32) + c2b_ref[...]


def _final(hs, batch3, jkWs, jk_b, c1_W, c1_b, c2_W, c2_b):
    hspec = pl.BlockSpec((_PB, _D), lambda i: (i, 0))
    return pl.pallas_call(
        _final_body,
        grid=(_NPB,),
        in_specs=[
            hspec, hspec, hspec, hspec, hspec,
            pl.BlockSpec((1, 1, _PB), lambda i: (i, 0, 0)),
            pl.BlockSpec((_L, _D, _D), lambda i: (0, 0, 0)),
            pl.BlockSpec((1, _D), lambda i: (0, 0)),
            pl.BlockSpec((_D, _D), lambda i: (0, 0)),
            pl.BlockSpec((1, _D), lambda i: (0, 0)),
            pl.BlockSpec((_D, _OUT), lambda i: (0, 0)),
            pl.BlockSpec((1, _OUT), lambda i: (0, 0)),
        ],
        out_specs=pl.BlockSpec((_G, _OUT), lambda i: (0, 0)),
        out_shape=jax.ShapeDtypeStruct((_G, _OUT), jnp.float32),
        scratch_shapes=[pltpu.VMEM((_G, _D), jnp.float32)],
    )(*hs, batch3, jkWs, jk_b.reshape(1, _D), c1_W, c1_b.reshape(1, _D),
      c2_W, c2_b.reshape(1, _OUT))


def kernel(x, edge_index, batch, Ws1, bs1, Ws2, bs2, jk_W, jk_b,
           c1_W, c1_b, c2_W, c2_b):
    src = edge_index[0]
    dst = edge_index[1]
    h = x
    hs = []
    for i in range(_L):
        agg2 = _seg_sum_sc(h, src, dst)
        h = _mlp(h, agg2, Ws1[i], bs1[i], Ws2[i], bs2[i])
        hs.append(h)
    jkWs = jk_W.reshape(_L, _D, _D)
    batch3 = batch.reshape(_NPB, 1, _PB)
    return _final(hs, batch3, jkWs, jk_b, c1_W, c1_b, c2_W, c2_b)


# speedup vs baseline: 12.4571x; 1.0100x over previous
import functools

import jax
import jax.numpy as jnp
from jax import lax
from jax.experimental import pallas as pl
from jax.experimental.pallas import tpu as pltpu
from jax.experimental.pallas import tpu_sc as plsc

_N = 10000
_E = 320000
_D = 128
_L = 5
_G = 128
_OUT = 16

_NC = 2
_NS = 16
_NW = _NC * _NS
_EPW = _E // _NW
_CH = 128
_CPS = 1
_SUP = _CPS * _CH
_NSUP = _EPW // _SUP
_TAIL = _EPW - _NSUP * _SUP
_NPAD = 10240
_RPT = _NPAD // _NS
_ZR = 80

_mesh = plsc.VectorSubcoreMesh(core_axis_name="c", subcore_axis_name="s")


@functools.partial(
    pl.kernel,
    mesh=_mesh,
    out_type=jax.ShapeDtypeStruct((_NC, _NPAD, _D), jnp.float32),
    scratch_types=[
        pltpu.VMEM((3, _CPS, _CH), jnp.int32),
        pltpu.VMEM((3, _CPS, _CH), jnp.int32),
        pltpu.VMEM((2, _SUP, _D), jnp.float32),
        pltpu.VMEM((_TAIL,), jnp.int32),
        pltpu.VMEM((_TAIL,), jnp.int32),
        pltpu.VMEM((_TAIL, _D), jnp.float32),
        pltpu.VMEM((_ZR, _D), jnp.float32),
        pltpu.VMEM_SHARED((_NPAD, _D), jnp.float32),
        pltpu.SemaphoreType.DMA,
        pltpu.SemaphoreType.DMA,
        pltpu.SemaphoreType.DMA,
    ],
)
def _seg_sum_sc(h_hbm, src_hbm, dst_hbm, out_hbm,
                sidx, didx, rows, sidx_t, didx_t, rows_t, zbuf, acc,
                sem_i, sem_g, sem_s):
    cid = lax.axis_index("c")
    sid = lax.axis_index("s")
    wid = sid * _NC + cid
    base = wid * _EPW

    def _fire_idx(s):
        b = s % 3
        off = base + s * _SUP
        for j in range(_CPS):
            pltpu.async_copy(src_hbm.at[pl.ds(off + j * _CH, _CH)],
                             sidx.at[b, j], sem_i)
            pltpu.async_copy(dst_hbm.at[pl.ds(off + j * _CH, _CH)],
                             didx.at[b, j], sem_i)

    def _wait_idx(s):
        b = s % 3
        off = base + s * _SUP
        for j in range(_CPS):
            pltpu.make_async_copy(src_hbm.at[pl.ds(off + j * _CH, _CH)],
                                  sidx.at[b, j], sem_i).wait()
            pltpu.make_async_copy(dst_hbm.at[pl.ds(off + j * _CH, _CH)],
                                  didx.at[b, j], sem_i).wait()

    def _fire_gather(s):
        b3, b2 = s % 3, s % 2
        for j in range(_CPS):
            pltpu.async_copy(h_hbm.at[sidx.at[b3, j]],
                             rows.at[b2, pl.ds(j * _CH, _CH)], sem_g)

    def _wait_gather(s):
        b3, b2 = s % 3, s % 2
        for j in range(_CPS):
            pltpu.make_async_copy(h_hbm.at[sidx.at[b3, j]],
                                  rows.at[b2, pl.ds(j * _CH, _CH)],
                                  sem_g).wait()

    def _fire_scatter(s):
        b3, b2 = s % 3, s % 2
        for j in range(_CPS):
            pltpu.async_copy(rows.at[b2, pl.ds(j * _CH, _CH)],
                             acc.at[didx.at[b3, j]], sem_s, add=True)

    def _wait_scatter(s):
        b3, b2 = s % 3, s % 2
        for j in range(_CPS):
            pltpu.make_async_copy(rows.at[b2, pl.ds(j * _CH, _CH)],
                                  acc.at[didx.at[b3, j]], sem_s).wait()

    _fire_idx(0)
    _fire_idx(1)

    zero = jnp.zeros((16,), jnp.float32)

    def _zrow(r, carry):
        for c in range(_D // 16):
            zbuf[r, pl.ds(c * 16, 16)] = zero
        return carry

    lax.fori_loop(0, _ZR, _zrow, 0)
    for k in range(_RPT // _ZR):
        pltpu.async_copy(zbuf, acc.at[pl.ds(sid * _RPT + k * _ZR, _ZR)], sem_s)

    _wait_idx(0)
    _fire_gather(0)
    for k in range(_RPT // _ZR):
        pltpu.make_async_copy(zbuf, acc.at[pl.ds(sid * _RPT + k * _ZR, _ZR)],
                              sem_s).wait()
    plsc.subcore_barrier()

    def _body(s, carry):
        @pl.when(s >= 1)
        def _():
            _wait_scatter(s - 1)

        @pl.when(s + 2 <= _NSUP - 1)
        def _():
            _fire_idx(s + 2)

        @pl.when(s + 1 <= _NSUP - 1)
        def _():
            _wait_idx(s + 1)
            _fire_gather(s + 1)

        _wait_gather(s)
        _fire_scatter(s)
        return carry

    lax.fori_loop(0, _NSUP, _body, 0)
    _wait_scatter(_NSUP - 1)

    offt = base + _NSUP * _SUP
    pltpu.sync_copy(src_hbm.at[pl.ds(offt, _TAIL)], sidx_t)
    pltpu.sync_copy(dst_hbm.at[pl.ds(offt, _TAIL)], didx_t)
    pltpu.async_copy(h_hbm.at[sidx_t], rows_t, sem_g).wait()
    pltpu.sync_copy(rows_t, acc.at[didx_t], add=True)

    plsc.subcore_barrier()
    pltpu.sync_copy(acc.at[pl.ds(sid * _RPT, _RPT)],
                    out_hbm.at[cid, pl.ds(sid * _RPT, _RPT)])


_MB = 2000
_NMB = _N // _MB


def _mlp_body(h_ref, a0_ref, a1_ref, w1_ref, b1_ref, w2_ref, b2_ref, o_ref):
    z = h_ref[...] + a0_ref[0] + a1_ref[0]
    z = jnp.dot(z, w1_ref[...], preferred_element_type=jnp.float32) + b1_ref[...]
    z = jnp.maximum(z, 0.0)
    z = jnp.dot(z, w2_ref[...], preferred_element_type=jnp.float32) + b2_ref[...]
    o_ref[...] = jnp.maximum(z, 0.0)


def _mlp(h, agg2, w1, b1, w2, b2):
    return pl.pallas_call(
        _mlp_body,
        grid=(_NMB,),
        in_specs=[
            pl.BlockSpec((_MB, _D), lambda i: (i, 0)),
            pl.BlockSpec((1, _MB, _D), lambda i: (0, i, 0)),
            pl.BlockSpec((1, _MB, _D), lambda i: (1, i, 0)),
            pl.BlockSpec((_D, _D), lambda i: (0, 0)),
            pl.BlockSpec((1, _D), lambda i: (0, 0)),
            pl.BlockSpec((_D, _D), lambda i: (0, 0)),
            pl.BlockSpec((1, _D), lambda i: (0, 0)),
        ],
        out_specs=pl.BlockSpec((_MB, _D), lambda i: (i, 0)),
        out_shape=jax.ShapeDtypeStruct((_N, _D), jnp.float32),
    )(h, agg2, agg2, w1, b1.reshape(1, _D), w2, b2.reshape(1, _D))


_PB = 2000
_NPB = _N // _PB
_SCALE = 1.0 / (1.0 + 1e-5) ** 0.5


def _final_body(h_ref, a0_ref, a1_ref, w1_ref, b1_ref, w2_ref, b2_ref,
                h0_ref, h1_ref, h2_ref, h3_ref, b_ref, jkW_ref,
                jkb_ref, c1W_ref, c1b_ref, c2W_ref, c2b_ref, o_ref, pooled):
    i = pl.program_id(0)

    @pl.when(i == 0)
    def _init():
        pooled[...] = jnp.zeros_like(pooled)

    z = h_ref[...] + a0_ref[0] + a1_ref[0]
    z = jnp.dot(z, w1_ref[...], preferred_element_type=jnp.float32) + b1_ref[...]
    z = jnp.maximum(z, 0.0)
    z = jnp.dot(z, w2_ref[...], preferred_element_type=jnp.float32) + b2_ref[...]
    h4 = jnp.maximum(z, 0.0)

    hs = (h0_ref, h1_ref, h2_ref, h3_ref)
    hlin = jnp.dot(h4, jkW_ref[_L - 1], preferred_element_type=jnp.float32)
    for l in range(_L - 1):
        hlin += jnp.dot(hs[l][...], jkW_ref[l], preferred_element_type=jnp.float32)
    hlin += jkb_ref[...]

    bvec = b_ref[0, 0, :]
    mask = (bvec[None, :] == lax.broadcasted_iota(jnp.int32, (_G, _PB), 0))
    pooled[...] += jnp.dot(mask.astype(jnp.float32), hlin,
                           preferred_element_type=jnp.float32)

    p = pooled[...]
    c = jnp.maximum((jnp.dot(p, c1W_ref[...], preferred_element_type=jnp.float32)
                     + c1b_ref[...]) * _SCALE, 0.0)
    o_ref[...] = jnp.dot(c, c2W_ref[...], preferred_element_type=jnp.float32) + c2b_ref[...]


def _final(h, agg2, w1, b1, w2, b2, hs, batch3, jkWs, jk_b,
           c1_W, c1_b, c2_W, c2_b):
    hspec = pl.BlockSpec((_PB, _D), lambda i: (i, 0))
    wspec = pl.BlockSpec((_D, _D), lambda i: (0, 0))
    bspec = pl.BlockSpec((1, _D), lambda i: (0, 0))
    return pl.pallas_call(
        _final_body,
        grid=(_NPB,),
        in_specs=[
            hspec,
            pl.BlockSpec((1, _PB, _D), lambda i: (0, i, 0)),
            pl.BlockSpec((1, _PB, _D), lambda i: (1, i, 0)),
            wspec, bspec, wspec, bspec,
            hspec, hspec, hspec, hspec,
            pl.BlockSpec((1, 1, _PB), lambda i: (i, 0, 0)),
            pl.BlockSpec((_L, _D, _D), lambda i: (0, 0, 0)),
            bspec, wspec, bspec,
            pl.BlockSpec((_D, _OUT), lambda i: (0, 0)),
            pl.BlockSpec((1, _OUT), lambda i: (0, 0)),
        ],
        out_specs=pl.BlockSpec((_G, _OUT), lambda i: (0, 0)),
        out_shape=jax.ShapeDtypeStruct((_G, _OUT), jnp.float32),
        scratch_shapes=[pltpu.VMEM((_G, _D), jnp.float32)],
    )(h, agg2, agg2, w1, b1.reshape(1, _D), w2, b2.reshape(1, _D),
      *hs, batch3, jkWs, jk_b.reshape(1, _D), c1_W, c1_b.reshape(1, _D),
      c2_W, c2_b.reshape(1, _OUT))


def kernel(x, edge_index, batch, Ws1, bs1, Ws2, bs2, jk_W, jk_b,
           c1_W, c1_b, c2_W, c2_b):
    src = edge_index[0]
    dst = edge_index[1]
    h = x
    hs = []
    for i in range(_L - 1):
        agg2 = _seg_sum_sc(h, src, dst)
        h = _mlp(h, agg2, Ws1[i], bs1[i], Ws2[i], bs2[i])
        hs.append(h)
    agg2 = _seg_sum_sc(h, src, dst)
    jkWs = jk_W.reshape(_L, _D, _D)
    batch3 = batch.reshape(_NPB, 1, _PB)
    return _final(h, agg2, Ws1[_L - 1], bs1[_L - 1], Ws2[_L - 1], bs2[_L - 1],
                  hs, batch3, jkWs, jk_b, c1_W, c1_b, c2_W, c2_b)


# speedup vs baseline: 12.8400x; 1.0307x over previous
import functools

import jax
import jax.numpy as jnp
from jax import lax
from jax.experimental import pallas as pl
from jax.experimental.pallas import tpu as pltpu
from jax.experimental.pallas import tpu_sc as plsc

_N = 10000
_E = 320000
_D = 128
_L = 5
_G = 128
_OUT = 16

_NC = 2
_NS = 16
_NW = _NC * _NS
_CH = 128
_SUP = _CH
_TCH = _E // _CH
_NSUP = _TCH // _NW
_NEXTRA = _TCH - _NSUP * _NW
_NPAD = 10240
_RPT = _NPAD // _NS
_ZR = 80

_mesh = plsc.VectorSubcoreMesh(core_axis_name="c", subcore_axis_name="s")


@functools.partial(
    pl.kernel,
    mesh=_mesh,
    out_type=jax.ShapeDtypeStruct((_NC, _NPAD, _D), jnp.float32),
    scratch_types=[
        pltpu.VMEM((3, 2, _CH), jnp.int32),
        pltpu.VMEM((2, _SUP, _D), jnp.float32),
        pltpu.VMEM((_ZR, _D), jnp.float32),
        pltpu.VMEM_SHARED((_NPAD, _D), jnp.float32),
        pltpu.SemaphoreType.DMA,
        pltpu.SemaphoreType.DMA,
        pltpu.SemaphoreType.DMA,
    ],
)
def _seg_sum_sc(h_hbm, e_hbm, out_hbm,
                eidx, rows, zbuf, acc,
                sem_i, sem_g, sem_s):
    cid = lax.axis_index("c")
    sid = lax.axis_index("s")
    wid = sid * _NC + cid
    nsup = _NSUP + jnp.where(wid < _NEXTRA, 1, 0)

    def _fire_idx(s):
        off = (wid + s * _NW) * _CH
        pltpu.async_copy(e_hbm.at[:, pl.ds(off, _CH)], eidx.at[s % 3], sem_i)

    def _wait_idx(s):
        off = (wid + s * _NW) * _CH
        pltpu.make_async_copy(e_hbm.at[:, pl.ds(off, _CH)], eidx.at[s % 3],
                              sem_i).wait()

    def _fire_gather(s):
        b3, b2 = s % 3, s % 2
        pltpu.async_copy(h_hbm.at[eidx.at[b3, 0]],
                         rows.at[b2, pl.ds(0, _CH)], sem_g)

    def _wait_gather(s):
        b3, b2 = s % 3, s % 2
        pltpu.make_async_copy(h_hbm.at[eidx.at[b3, 0]],
                              rows.at[b2, pl.ds(0, _CH)], sem_g).wait()

    def _fire_scatter(s):
        b3, b2 = s % 3, s % 2
        pltpu.async_copy(rows.at[b2, pl.ds(0, _CH)],
                         acc.at[eidx.at[b3, 1]], sem_s, add=True)

    def _wait_scatter(s):
        b3, b2 = s % 3, s % 2
        pltpu.make_async_copy(rows.at[b2, pl.ds(0, _CH)],
                              acc.at[eidx.at[b3, 1]], sem_s).wait()

    _fire_idx(0)
    _fire_idx(1)

    zero = jnp.zeros((16,), jnp.float32)

    def _zrow(r, carry):
        for c in range(_D // 16):
            zbuf[r, pl.ds(c * 16, 16)] = zero
        return carry

    lax.fori_loop(0, _ZR, _zrow, 0)
    for k in range(_RPT // _ZR):
        pltpu.async_copy(zbuf, acc.at[pl.ds(sid * _RPT + k * _ZR, _ZR)], sem_s)

    _wait_idx(0)
    _fire_gather(0)
    for k in range(_RPT // _ZR):
        pltpu.make_async_copy(zbuf, acc.at[pl.ds(sid * _RPT + k * _ZR, _ZR)],
                              sem_s).wait()
    plsc.subcore_barrier()

    def _body(s, carry):
        @pl.when(s >= 1)
        def _():
            _wait_scatter(s - 1)

        @pl.when(s + 2 <= nsup - 1)
        def _():
            _fire_idx(s + 2)

        @pl.when(s + 1 <= nsup - 1)
        def _():
            _wait_idx(s + 1)
            _fire_gather(s + 1)

        _wait_gather(s)
        _fire_scatter(s)
        return carry

    lax.fori_loop(0, nsup, _body, 0)
    _wait_scatter(nsup - 1)

    plsc.subcore_barrier()
    pltpu.sync_copy(acc.at[pl.ds(sid * _RPT, _RPT)],
                    out_hbm.at[cid, pl.ds(sid * _RPT, _RPT)])


_MB = 2000
_NMB = _N // _MB


def _mlp_body(h_ref, a0_ref, a1_ref, w1_ref, b1_ref, w2_ref, b2_ref, o_ref):
    z = h_ref[...] + a0_ref[0] + a1_ref[0]
    z = jnp.dot(z, w1_ref[...], preferred_element_type=jnp.float32) + b1_ref[...]
    z = jnp.maximum(z, 0.0)
    z = jnp.dot(z, w2_ref[...], preferred_element_type=jnp.float32) + b2_ref[...]
    o_ref[...] = jnp.maximum(z, 0.0)


def _mlp(h, agg2, w1, b1, w2, b2):
    return pl.pallas_call(
        _mlp_body,
        grid=(_NMB,),
        in_specs=[
            pl.BlockSpec((_MB, _D), lambda i: (i, 0)),
            pl.BlockSpec((1, _MB, _D), lambda i: (0, i, 0)),
            pl.BlockSpec((1, _MB, _D), lambda i: (1, i, 0)),
            pl.BlockSpec((_D, _D), lambda i: (0, 0)),
            pl.BlockSpec((1, _D), lambda i: (0, 0)),
            pl.BlockSpec((_D, _D), lambda i: (0, 0)),
            pl.BlockSpec((1, _D), lambda i: (0, 0)),
        ],
        out_specs=pl.BlockSpec((_MB, _D), lambda i: (i, 0)),
        out_shape=jax.ShapeDtypeStruct((_N, _D), jnp.float32),
    )(h, agg2, agg2, w1, b1.reshape(1, _D), w2, b2.reshape(1, _D))


_PB = 2000
_NPB = _N // _PB
_SCALE = 1.0 / (1.0 + 1e-5) ** 0.5


def _final_body(h_ref, a0_ref, a1_ref, w1_ref, b1_ref, w2_ref, b2_ref,
                h0_ref, h1_ref, h2_ref, h3_ref, b_ref, jkW_ref,
                jkb_ref, c1W_ref, c1b_ref, c2W_ref, c2b_ref, o_ref, pooled):
    i = pl.program_id(0)

    @pl.when(i == 0)
    def _init():
        pooled[...] = jnp.zeros_like(pooled)

    z = h_ref[...] + a0_ref[0] + a1_ref[0]
    z = jnp.dot(z, w1_ref[...], preferred_element_type=jnp.float32) + b1_ref[...]
    z = jnp.maximum(z, 0.0)
    z = jnp.dot(z, w2_ref[...], preferred_element_type=jnp.float32) + b2_ref[...]
    h4 = jnp.maximum(z, 0.0)

    hs = (h0_ref, h1_ref, h2_ref, h3_ref)
    hlin = jnp.dot(h4, jkW_ref[_L - 1], preferred_element_type=jnp.float32)
    for l in range(_L - 1):
        hlin += jnp.dot(hs[l][...], jkW_ref[l], preferred_element_type=jnp.float32)
    hlin += jkb_ref[...]

    bvec = b_ref[0, 0, :]
    mask = (bvec[None, :] == lax.broadcasted_iota(jnp.int32, (_G, _PB), 0))
    pooled[...] += jnp.dot(mask.astype(jnp.float32), hlin,
                           preferred_element_type=jnp.float32)

    p = pooled[...]
    c = jnp.maximum((jnp.dot(p, c1W_ref[...], preferred_element_type=jnp.float32)
                     + c1b_ref[...]) * _SCALE, 0.0)
    o_ref[...] = jnp.dot(c, c2W_ref[...], preferred_element_type=jnp.float32) + c2b_ref[...]


def _final(h, agg2, w1, b1, w2, b2, hs, batch3, jkWs, jk_b,
           c1_W, c1_b, c2_W, c2_b):
    hspec = pl.BlockSpec((_PB, _D), lambda i: (i, 0))
    wspec = pl.BlockSpec((_D, _D), lambda i: (0, 0))
    bspec = pl.BlockSpec((1, _D), lambda i: (0, 0))
    return pl.pallas_call(
        _final_body,
        grid=(_NPB,),
        in_specs=[
            hspec,
            pl.BlockSpec((1, _PB, _D), lambda i: (0, i, 0)),
            pl.BlockSpec((1, _PB, _D), lambda i: (1, i, 0)),
            wspec, bspec, wspec, bspec,
            hspec, hspec, hspec, hspec,
            pl.BlockSpec((1, 1, _PB), lambda i: (i, 0, 0)),
            pl.BlockSpec((_L, _D, _D), lambda i: (0, 0, 0)),
            bspec, wspec, bspec,
            pl.BlockSpec((_D, _OUT), lambda i: (0, 0)),
            pl.BlockSpec((1, _OUT), lambda i: (0, 0)),
        ],
        out_specs=pl.BlockSpec((_G, _OUT), lambda i: (0, 0)),
        out_shape=jax.ShapeDtypeStruct((_G, _OUT), jnp.float32),
        scratch_shapes=[pltpu.VMEM((_G, _D), jnp.float32)],
    )(h, agg2, agg2, w1, b1.reshape(1, _D), w2, b2.reshape(1, _D),
      *hs, batch3, jkWs, jk_b.reshape(1, _D), c1_W, c1_b.reshape(1, _D),
      c2_W, c2_b.reshape(1, _OUT))


def kernel(x, edge_index, batch, Ws1, bs1, Ws2, bs2, jk_W, jk_b,
           c1_W, c1_b, c2_W, c2_b):
    h = x
    hs = []
    for i in range(_L - 1):
        agg2 = _seg_sum_sc(h, edge_index)
        h = _mlp(h, agg2, Ws1[i], bs1[i], Ws2[i], bs2[i])
        hs.append(h)
    agg2 = _seg_sum_sc(h, edge_index)
    jkWs = jk_W.reshape(_L, _D, _D)
    batch3 = batch.reshape(_NPB, 1, _PB)
    return _final(h, agg2, Ws1[_L - 1], bs1[_L - 1], Ws2[_L - 1], bs2[_L - 1],
                  hs, batch3, jkWs, jk_b, c1_W, c1_b, c2_W, c2_b)
